# Initial kernel scaffold; baseline (speedup 1.0000x reference)
import jax
import jax.numpy as jnp
from jax.experimental import pallas as pl


def kernel(x, edgeIndex, W1_l, b1_l, W1_r, W2_l, b2_l, W2_r):
    raise NotImplementedError("write your pallas kernel here")


# speedup vs baseline: 3.5149x; 3.5149x over previous
import functools

import jax
import jax.numpy as jnp
from jax import lax
from jax.experimental import pallas as pl
from jax.experimental.pallas import tpu as pltpu
from jax.experimental.pallas import tpu_sc as plsc

N = 10000
D = 128
E = 320000
CNT_LANES = 16
W_AUG = D + CNT_LANES

NC = 2
NS = 16
NW = NC * NS
CHUNK = 128
K = -(-E // (NW * CHUNK))
K = K + (K % 2)
E_PAD = NW * CHUNK * K
N_ACC = 10112
ROWS_PER_TILE = N_ACC // NS


def _sc_agg_body(table, src_hbm, dst_hbm, out, src_v, dst_v, rows_v, acc, sem):
    c = lax.axis_index("c")
    s = lax.axis_index("s")
    wid = s * NC + c

    @pl.loop(0, CHUNK)
    def _(i):
        @pl.loop(0, W_AUG, step=16)
        def _(j):
            rows_v[i, pl.ds(j, 16)] = jnp.zeros((16,), jnp.float32)

    base = s * ROWS_PER_TILE
    @pl.loop(0, 4)
    def _(q):
        pltpu.sync_copy(rows_v, acc.at[pl.ds(base + q * CHUNK, CHUNK)])
    pltpu.sync_copy(rows_v.at[pl.ds(0, 120)],
                    acc.at[pl.ds(base + 4 * CHUNK, 120)])

    pltpu.sync_copy(src_hbm.at[wid], src_v)
    pltpu.sync_copy(dst_hbm.at[wid], dst_v)

    plsc.subcore_barrier()

    @pl.loop(0, K)
    def _(j):
        pltpu.async_copy(table.at[src_v.at[j]], rows_v, sem).wait()
        pltpu.sync_copy(rows_v, acc.at[dst_v.at[j]], add=True)

    plsc.subcore_barrier()

    pltpu.sync_copy(acc.at[pl.ds(base, ROWS_PER_TILE)],
                    out.at[c, pl.ds(base, ROWS_PER_TILE)])


@jax.jit
def _sc_agg(table, src_hbm, dst_hbm):
    mesh = plsc.VectorSubcoreMesh(core_axis_name="c", subcore_axis_name="s")
    kfn = pl.kernel(
        _sc_agg_body,
        out_type=jax.ShapeDtypeStruct((NC, N_ACC, W_AUG), jnp.float32),
        mesh=mesh,
        scratch_types=[
            pltpu.VMEM((K, CHUNK), jnp.int32),
            pltpu.VMEM((K, CHUNK), jnp.int32),
            pltpu.VMEM((CHUNK, W_AUG), jnp.float32),
            pltpu.VMEM_SHARED((N_ACC, W_AUG), jnp.float32),
            pltpu.SemaphoreType.DMA,
        ],
        compiler_params=pltpu.CompilerParams(use_tc_tiling_on_sc=False),
    )
    return kfn(table, src_hbm, dst_hbm)


def _tc_pre_body(x_ref, wl_ref, wr_ref, b_ref, yaug_ref, r_ref):
    xv = x_ref[...]
    yaug_ref[:, :D] = jnp.dot(xv, wl_ref[...], preferred_element_type=jnp.float32)
    yaug_ref[:, D:] = jnp.ones((N, CNT_LANES), jnp.float32)
    r_ref[...] = jnp.dot(xv, wr_ref[...], preferred_element_type=jnp.float32) + b_ref[...]


@jax.jit
def _tc_pre(x, wl_t, wr_t, b):
    return pl.pallas_call(
        _tc_pre_body,
        out_shape=(jax.ShapeDtypeStruct((N, W_AUG), jnp.float32),
                   jax.ShapeDtypeStruct((N, D), jnp.float32)),
    )(x, wl_t, wr_t, b)


def _mean_from_partials(p_ref):
    ssum = p_ref[0, :N, :D] + p_ref[1, :N, :D]
    cnt16 = p_ref[0, :N, D:] + p_ref[1, :N, D:]
    cnt = jnp.max(cnt16, axis=1, keepdims=True)
    return ssum / jnp.maximum(cnt, 1.0)


def _tc_mid_body(p_ref, r1_ref, wl_ref, wr_ref, b_ref, yaug_ref, r2_ref):
    h = jnp.maximum(_mean_from_partials(p_ref) + r1_ref[...], 0.0)
    yaug_ref[:, :D] = jnp.dot(h, wl_ref[...], preferred_element_type=jnp.float32)
    yaug_ref[:, D:] = jnp.ones((N, CNT_LANES), jnp.float32)
    r2_ref[...] = jnp.dot(h, wr_ref[...], preferred_element_type=jnp.float32) + b_ref[...]


@jax.jit
def _tc_mid(p1, r1, wl_t, wr_t, b):
    return pl.pallas_call(
        _tc_mid_body,
        out_shape=(jax.ShapeDtypeStruct((N, W_AUG), jnp.float32),
                   jax.ShapeDtypeStruct((N, D), jnp.float32)),
    )(p1, r1, wl_t, wr_t, b)


def _tc_post_body(p_ref, r2_ref, out_ref):
    out_ref[...] = _mean_from_partials(p_ref) + r2_ref[...]


@jax.jit
def _tc_post(p2, r2):
    return pl.pallas_call(
        _tc_post_body,
        out_shape=jax.ShapeDtypeStruct((N, D), jnp.float32),
    )(p2, r2)


def kernel(x, edgeIndex, W1_l, b1_l, W1_r, W2_l, b2_l, W2_r):
    src = edgeIndex[0]
    dst = edgeIndex[1]
    pad = E_PAD - E
    srcp = jnp.concatenate([src, jnp.zeros((pad,), src.dtype)]).reshape(NW, K, CHUNK)
    dstp = jnp.concatenate([dst, jnp.full((pad,), N, dst.dtype)]).reshape(NW, K, CHUNK)

    yaug1, r1 = _tc_pre(x, W1_l.T, W1_r.T, b1_l[None, :])
    p1 = _sc_agg(yaug1, srcp, dstp)
    yaug2, r2 = _tc_mid(p1, r1, W2_l.T, W2_r.T, b2_l[None, :])
    p2 = _sc_agg(yaug2, srcp, dstp)
    return _tc_post(p2, r2)


# speedup vs baseline: 3.7294x; 1.0610x over previous
import functools

import jax
import jax.numpy as jnp
from jax import lax
from jax.experimental import pallas as pl
from jax.experimental.pallas import tpu as pltpu
from jax.experimental.pallas import tpu_sc as plsc

N = 10000
D = 128
E = 320000
CNT_LANES = 16
W_AUG = D + CNT_LANES

NC = 2
NS = 16
NW = NC * NS
CHUNK = 64
K = -(-E // (NW * CHUNK))
K = K + (K % 2)
E_PAD = NW * CHUNK * K
N_ACC = 10112
ROWS_PER_TILE = N_ACC // NS


def _sc_agg_body(table, src_hbm, dst_hbm, out, src_v, dst_v, rows_a, rows_b,
                 acc, isem, gsem_a, gsem_b):
    c = lax.axis_index("c")
    s = lax.axis_index("s")
    wid = s * NC + c

    idx_cp = pltpu.async_copy(src_hbm.at[wid], src_v, isem)
    pltpu.async_copy(dst_hbm.at[wid], dst_v, isem)

    @pl.loop(0, CHUNK)
    def _(i):
        @pl.loop(0, W_AUG, step=16)
        def _(j):
            rows_a[i, pl.ds(j, 16)] = jnp.zeros((16,), jnp.float32)

    base = s * ROWS_PER_TILE
    nz = ROWS_PER_TILE // CHUNK
    rem = ROWS_PER_TILE - nz * CHUNK

    @pl.loop(0, nz)
    def _(q):
        pltpu.sync_copy(rows_a, acc.at[pl.ds(base + q * CHUNK, CHUNK)])
    if rem:
        pltpu.sync_copy(rows_a.at[pl.ds(0, rem)],
                        acc.at[pl.ds(base + nz * CHUNK, rem)])

    idx_cp.wait()
    pltpu.make_async_copy(dst_hbm.at[wid], dst_v, isem).wait()

    plsc.subcore_barrier()

    pltpu.async_copy(table.at[src_v.at[0]], rows_a, gsem_a)
    pltpu.async_copy(table.at[src_v.at[1]], rows_b, gsem_b)

    @pl.loop(0, K, step=2)
    def _(j):
        pltpu.make_async_copy(table.at[src_v.at[j]], rows_a, gsem_a).wait()
        pltpu.sync_copy(rows_a, acc.at[dst_v.at[j]], add=True)
        pltpu.async_copy(table.at[src_v.at[j + 2]], rows_a, gsem_a)
        pltpu.make_async_copy(table.at[src_v.at[j + 1]], rows_b, gsem_b).wait()
        pltpu.sync_copy(rows_b, acc.at[dst_v.at[j + 1]], add=True)
        pltpu.async_copy(table.at[src_v.at[j + 3]], rows_b, gsem_b)

    pltpu.make_async_copy(table.at[src_v.at[K]], rows_a, gsem_a).wait()
    pltpu.make_async_copy(table.at[src_v.at[K + 1]], rows_b, gsem_b).wait()

    plsc.subcore_barrier()

    pltpu.sync_copy(acc.at[pl.ds(base, ROWS_PER_TILE)],
                    out.at[c, pl.ds(base, ROWS_PER_TILE)])


@jax.jit
def _sc_agg(table, src_hbm, dst_hbm):
    mesh = plsc.VectorSubcoreMesh(core_axis_name="c", subcore_axis_name="s")
    kfn = pl.kernel(
        _sc_agg_body,
        out_type=jax.ShapeDtypeStruct((NC, N_ACC, W_AUG), jnp.float32),
        mesh=mesh,
        scratch_types=[
            pltpu.VMEM((K + 2, CHUNK), jnp.int32),
            pltpu.VMEM((K, CHUNK), jnp.int32),
            pltpu.VMEM((CHUNK, W_AUG), jnp.float32),
            pltpu.VMEM((CHUNK, W_AUG), jnp.float32),
            pltpu.VMEM_SHARED((N_ACC, W_AUG), jnp.float32),
            pltpu.SemaphoreType.DMA,
            pltpu.SemaphoreType.DMA,
            pltpu.SemaphoreType.DMA,
        ],
        compiler_params=pltpu.CompilerParams(use_tc_tiling_on_sc=False),
    )
    return kfn(table, src_hbm, dst_hbm)


def _tc_pre_body(x_ref, wl_ref, wr_ref, b_ref, yaug_ref, r_ref):
    xv = x_ref[...]
    yaug_ref[:, :D] = jnp.dot(xv, wl_ref[...], preferred_element_type=jnp.float32)
    yaug_ref[:, D:] = jnp.ones((N, CNT_LANES), jnp.float32)
    r_ref[...] = jnp.dot(xv, wr_ref[...], preferred_element_type=jnp.float32) + b_ref[...]


@jax.jit
def _tc_pre(x, wl_t, wr_t, b):
    return pl.pallas_call(
        _tc_pre_body,
        out_shape=(jax.ShapeDtypeStruct((N, W_AUG), jnp.float32),
                   jax.ShapeDtypeStruct((N, D), jnp.float32)),
    )(x, wl_t, wr_t, b)


def _mean_from_partials(p_ref):
    ssum = p_ref[0, :N, :D] + p_ref[1, :N, :D]
    cnt16 = p_ref[0, :N, D:] + p_ref[1, :N, D:]
    cnt = jnp.max(cnt16, axis=1, keepdims=True)
    return ssum / jnp.maximum(cnt, 1.0)


def _tc_mid_body(p_ref, r1_ref, wl_ref, wr_ref, b_ref, yaug_ref, r2_ref):
    h = jnp.maximum(_mean_from_partials(p_ref) + r1_ref[...], 0.0)
    yaug_ref[:, :D] = jnp.dot(h, wl_ref[...], preferred_element_type=jnp.float32)
    yaug_ref[:, D:] = jnp.ones((N, CNT_LANES), jnp.float32)
    r2_ref[...] = jnp.dot(h, wr_ref[...], preferred_element_type=jnp.float32) + b_ref[...]


@jax.jit
def _tc_mid(p1, r1, wl_t, wr_t, b):
    return pl.pallas_call(
        _tc_mid_body,
        out_shape=(jax.ShapeDtypeStruct((N, W_AUG), jnp.float32),
                   jax.ShapeDtypeStruct((N, D), jnp.float32)),
    )(p1, r1, wl_t, wr_t, b)


def _tc_post_body(p_ref, r2_ref, out_ref):
    out_ref[...] = _mean_from_partials(p_ref) + r2_ref[...]


@jax.jit
def _tc_post(p2, r2):
    return pl.pallas_call(
        _tc_post_body,
        out_shape=jax.ShapeDtypeStruct((N, D), jnp.float32),
    )(p2, r2)


def kernel(x, edgeIndex, W1_l, b1_l, W1_r, W2_l, b2_l, W2_r):
    src = edgeIndex[0]
    dst = edgeIndex[1]
    pad = E_PAD - E
    srcp = jnp.concatenate([src, jnp.zeros((pad,), src.dtype)]).reshape(NW, K, CHUNK)
    srcp = jnp.concatenate([srcp, jnp.zeros((NW, 2, CHUNK), src.dtype)], axis=1)
    dstp = jnp.concatenate([dst, jnp.full((pad,), N, dst.dtype)]).reshape(NW, K, CHUNK)

    yaug1, r1 = _tc_pre(x, W1_l.T, W1_r.T, b1_l[None, :])
    p1 = _sc_agg(yaug1, srcp, dstp)
    yaug2, r2 = _tc_mid(p1, r1, W2_l.T, W2_r.T, b2_l[None, :])
    p2 = _sc_agg(yaug2, srcp, dstp)
    return _tc_post(p2, r2)


# speedup vs baseline: 9.5200x; 2.5527x over previous
import functools

import jax
import jax.numpy as jnp
from jax import lax
from jax.experimental import pallas as pl
from jax.experimental.pallas import tpu as pltpu
from jax.experimental.pallas import tpu_sc as plsc

N = 10000
D = 128
E = 320000
CNT_LANES = 16
W_AUG = D + CNT_LANES

NC = 2
NS = 16
NW = NC * NS
CHUNK = 64
K = -(-E // (NW * CHUNK))
K = K + (K % 2)
E_PAD = NW * CHUNK * K
N_ACC = 10112
ROWS_PER_TILE = N_ACC // NS


def _sc_agg_body(table, src_hbm, dst_hbm, out, src_v, dst_v, rows_a, rows_b,
                 acc, isem, gsem_a, gsem_b):
    c = lax.axis_index("c")
    s = lax.axis_index("s")
    wid = s * NC + c

    idx_cp = pltpu.async_copy(src_hbm.at[wid], src_v, isem)
    pltpu.async_copy(dst_hbm.at[wid], dst_v, isem)

    @pl.loop(0, CHUNK)
    def _(i):
        @pl.loop(0, W_AUG, step=16)
        def _(j):
            rows_a[i, pl.ds(j, 16)] = jnp.zeros((16,), jnp.float32)

    base = s * ROWS_PER_TILE
    nz = ROWS_PER_TILE // CHUNK
    rem = ROWS_PER_TILE - nz * CHUNK

    @pl.loop(0, nz)
    def _(q):
        pltpu.sync_copy(rows_a, acc.at[pl.ds(base + q * CHUNK, CHUNK)])
    if rem:
        pltpu.sync_copy(rows_a.at[pl.ds(0, rem)],
                        acc.at[pl.ds(base + nz * CHUNK, rem)])

    idx_cp.wait()
    pltpu.make_async_copy(dst_hbm.at[wid], dst_v, isem).wait()

    plsc.subcore_barrier()

    pltpu.async_copy(table.at[src_v.at[0]], rows_a, gsem_a)
    pltpu.async_copy(table.at[src_v.at[1]], rows_b, gsem_b)

    @pl.loop(0, K, step=2)
    def _(j):
        pltpu.make_async_copy(table.at[src_v.at[j]], rows_a, gsem_a).wait()
        pltpu.sync_copy(rows_a, acc.at[dst_v.at[j]], add=True)
        pltpu.async_copy(table.at[src_v.at[j + 2]], rows_a, gsem_a)
        pltpu.make_async_copy(table.at[src_v.at[j + 1]], rows_b, gsem_b).wait()
        pltpu.sync_copy(rows_b, acc.at[dst_v.at[j + 1]], add=True)
        pltpu.async_copy(table.at[src_v.at[j + 3]], rows_b, gsem_b)

    pltpu.make_async_copy(table.at[src_v.at[K]], rows_a, gsem_a).wait()
    pltpu.make_async_copy(table.at[src_v.at[K + 1]], rows_b, gsem_b).wait()

    plsc.subcore_barrier()

    pltpu.sync_copy(acc.at[pl.ds(base, ROWS_PER_TILE)],
                    out.at[c, pl.ds(base, ROWS_PER_TILE)])


@jax.jit
def _sc_agg(table, src_hbm, dst_hbm):
    mesh = plsc.VectorSubcoreMesh(core_axis_name="c", subcore_axis_name="s")
    kfn = pl.kernel(
        _sc_agg_body,
        out_type=jax.ShapeDtypeStruct((NC, N_ACC, W_AUG), jnp.float32),
        mesh=mesh,
        scratch_types=[
            pltpu.VMEM((K + 2, CHUNK), jnp.int32),
            pltpu.VMEM((K, CHUNK), jnp.int32),
            pltpu.VMEM((CHUNK, W_AUG), jnp.float32),
            pltpu.VMEM((CHUNK, W_AUG), jnp.float32),
            pltpu.VMEM_SHARED((N_ACC, W_AUG), jnp.float32),
            pltpu.SemaphoreType.DMA,
            pltpu.SemaphoreType.DMA,
            pltpu.SemaphoreType.DMA,
        ],
        compiler_params=pltpu.CompilerParams(use_tc_tiling_on_sc=False),
    )
    return kfn(table, src_hbm, dst_hbm)


def _tc_pre_body(x_ref, wl_ref, wr_ref, b_ref, yaug_ref, r_ref):
    xv = x_ref[...]
    yaug_ref[:, :D] = jnp.dot(xv, wl_ref[...], preferred_element_type=jnp.float32)
    yaug_ref[:, D:] = jnp.ones((N, CNT_LANES), jnp.float32)
    r_ref[...] = jnp.dot(xv, wr_ref[...], preferred_element_type=jnp.float32) + b_ref[...]


@jax.jit
def _tc_pre(x, wl_t, wr_t, b):
    return pl.pallas_call(
        _tc_pre_body,
        out_shape=(jax.ShapeDtypeStruct((N, W_AUG), jnp.float32),
                   jax.ShapeDtypeStruct((N, D), jnp.float32)),
    )(x, wl_t, wr_t, b)


def _mean_from_partials(p_ref):
    ssum = p_ref[0, :N, :D] + p_ref[1, :N, :D]
    cnt16 = p_ref[0, :N, D:] + p_ref[1, :N, D:]
    cnt = jnp.max(cnt16, axis=1, keepdims=True)
    return ssum / jnp.maximum(cnt, 1.0)


def _tc_mid_body(p_ref, r1_ref, wl_ref, wr_ref, b_ref, yaug_ref, r2_ref):
    h = jnp.maximum(_mean_from_partials(p_ref) + r1_ref[...], 0.0)
    yaug_ref[:, :D] = jnp.dot(h, wl_ref[...], preferred_element_type=jnp.float32)
    yaug_ref[:, D:] = jnp.ones((N, CNT_LANES), jnp.float32)
    r2_ref[...] = jnp.dot(h, wr_ref[...], preferred_element_type=jnp.float32) + b_ref[...]


@jax.jit
def _tc_mid(p1, r1, wl_t, wr_t, b):
    return pl.pallas_call(
        _tc_mid_body,
        out_shape=(jax.ShapeDtypeStruct((N, W_AUG), jnp.float32),
                   jax.ShapeDtypeStruct((N, D), jnp.float32)),
    )(p1, r1, wl_t, wr_t, b)


def _tc_post_body(p_ref, r2_ref, out_ref):
    out_ref[...] = _mean_from_partials(p_ref) + r2_ref[...]


@jax.jit
def _tc_post(p2, r2):
    return pl.pallas_call(
        _tc_post_body,
        out_shape=jax.ShapeDtypeStruct((N, D), jnp.float32),
    )(p2, r2)


def kernel(x, edgeIndex, W1_l, b1_l, W1_r, W2_l, b2_l, W2_r):
    src = edgeIndex[0]
    dst = edgeIndex[1]
    pad = E_PAD - E
    pad_src = (jnp.arange(pad, dtype=src.dtype) * 37) % N
    tail_src = (jnp.arange(NW * 2 * CHUNK, dtype=src.dtype) * 53) % N
    srcp = jnp.concatenate([src, pad_src]).reshape(NW, K, CHUNK)
    srcp = jnp.concatenate([srcp, tail_src.reshape(NW, 2, CHUNK)], axis=1)
    pad_dst = N + (jnp.arange(pad, dtype=dst.dtype) % (N_ACC - N))
    dstp = jnp.concatenate([dst, pad_dst]).reshape(NW, K, CHUNK)

    yaug1, r1 = _tc_pre(x, W1_l.T, W1_r.T, b1_l[None, :])
    p1 = _sc_agg(yaug1, srcp, dstp)
    yaug2, r2 = _tc_mid(p1, r1, W2_l.T, W2_r.T, b2_l[None, :])
    p2 = _sc_agg(yaug2, srcp, dstp)
    return _tc_post(p2, r2)
